# baseline (speedup 1.0000x reference)
import functools

import jax
import jax.numpy as jnp
from jax import lax
from jax.experimental import pallas as pl
from jax.experimental.pallas import tpu as pltpu
from jax.experimental.pallas import tpu_sc as plsc

B = 16384
D = 32
L = 16
NW = 32
BPW = B // NW


def _gather_body(uid_hbm, iid_hbm, ut_hbm, it_hbm, urows_hbm, irows_hbm,
                 uidx, iidx, urows, irows, sem_u, sem_i):
    nc = 2
    wid = lax.axis_index("s") * nc + lax.axis_index("c")
    base = wid * BPW

    pltpu.sync_copy(uid_hbm.at[pl.ds(base, BPW)], uidx)
    pltpu.sync_copy(iid_hbm.at[pl.ds(base, BPW)], iidx)

    cp_u = pltpu.async_copy(ut_hbm.at[uidx], urows, sem_u)
    cp_i = pltpu.async_copy(it_hbm.at[iidx], irows, sem_i)
    cp_u.wait()
    pltpu.sync_copy(urows, urows_hbm.at[pl.ds(base, BPW)])
    cp_i.wait()
    pltpu.sync_copy(irows, irows_hbm.at[pl.ds(base, BPW)])


def _tail_body(u_ref, i_ref, wb_ref, out_ref):
    p = u_ref[...] * i_ref[...]
    wt = wb_ref[0:1, 0:D]
    s = jnp.sum(p * wt, axis=1, keepdims=True)
    bias = wb_ref[0, D]
    out_ref[...] = 1.0 / (1.0 + jnp.exp(-(s + bias)))


@jax.jit
def _gmf(user_ids, item_ids, user_table, item_table, wb):
    mesh = plsc.VectorSubcoreMesh(core_axis_name="c", subcore_axis_name="s")
    gather = functools.partial(
        pl.kernel,
        mesh=mesh,
        out_type=(
            jax.ShapeDtypeStruct((B, D), jnp.float32),
            jax.ShapeDtypeStruct((B, D), jnp.float32),
        ),
        scratch_types=[
            pltpu.VMEM((BPW,), jnp.int32),
            pltpu.VMEM((BPW,), jnp.int32),
            pltpu.VMEM((BPW, D), jnp.float32),
            pltpu.VMEM((BPW, D), jnp.float32),
            pltpu.SemaphoreType.DMA,
            pltpu.SemaphoreType.DMA,
        ],
        compiler_params=pltpu.CompilerParams(use_tc_tiling_on_sc=False),
    )(_gather_body)
    urows, irows = gather(user_ids, item_ids, user_table, item_table)

    blk = 2048
    out = pl.pallas_call(
        _tail_body,
        grid=(B // blk,),
        in_specs=[
            pl.BlockSpec((blk, D), lambda i: (i, 0)),
            pl.BlockSpec((blk, D), lambda i: (i, 0)),
            pl.BlockSpec((1, 128), lambda i: (0, 0)),
        ],
        out_specs=pl.BlockSpec((blk, 1), lambda i: (i, 0)),
        out_shape=jax.ShapeDtypeStruct((B, 1), jnp.float32),
    )(urows, irows, wb)
    return out


def kernel(user_ids, item_ids, user_table, item_table, fc_w, fc_b):
    wb = jnp.concatenate(
        [fc_w.reshape(-1), fc_b.reshape(-1),
         jnp.zeros((128 - D - 1,), jnp.float32)]).reshape(1, 128)
    return _gmf(user_ids.astype(jnp.int32), item_ids.astype(jnp.int32),
                user_table, item_table, wb)


# speedup vs baseline: 3.9692x; 3.9692x over previous
import functools

import jax
import jax.numpy as jnp
from jax import lax
from jax.experimental import pallas as pl
from jax.experimental.pallas import tpu as pltpu
from jax.experimental.pallas import tpu_sc as plsc

B = 16384
D = 32
L = 16
NW = 32
BPW = B // NW
NFG = D // 8
GQ = 4
NG = BPW // GQ


def _gmf_body(uid_hbm, iid_hbm, ut_hbm, it_hbm, wb_hbm, res_hbm,
              uidx, iidx, ublk, iblk, wv, tbuf, outv, sem_u, sem_i):
    nc = 2
    wid = lax.axis_index("s") * nc + lax.axis_index("c")
    base = wid * BPW

    pltpu.sync_copy(uid_hbm.at[pl.ds(base, BPW)], uidx.at[pl.ds(0, BPW)])
    pltpu.sync_copy(iid_hbm.at[pl.ds(base, BPW)], iidx.at[pl.ds(0, BPW)])
    pltpu.sync_copy(wb_hbm, wv)

    lane = lax.broadcasted_iota(jnp.int32, (L,), 0)
    s_v = lane & 7
    fg_lo = lane >> 3
    fg_hi = fg_lo + 2
    w_lo = wv[pl.ds(0, L)]
    w_hi = wv[pl.ds(L, L)]
    bias = wv[pl.ds(2 * L, L)][0]

    def issue_id(j, slot, q, tab, blk, sem):
        co = pl.multiple_of((j >> 7) * 128, 128)
        for fg in range(NFG):
            pltpu.async_copy(tab.at[pl.ds(8 * fg, 8), pl.ds(co, 128)],
                             blk.at[slot, q, fg], sem)

    def idvecs(g):
        bb = pl.multiple_of((g >> 1) * 8, 8)
        return uidx[pl.ds(bb, L)], iidx[pl.ds(bb, L)]

    def issue_group(g, par):
        idvu, idvi = idvecs(g)
        for q in range(GQ):
            l = par * GQ + q
            issue_id(idvu[l], par, q, ut_hbm, ublk, sem_u)
            issue_id(idvi[l], par, q, it_hbm, iblk, sem_i)

    def drain_group():
        for _ in range(GQ * NFG):
            pltpu.make_async_copy(ut_hbm.at[pl.ds(0, 8), pl.ds(0, 128)],
                                  ublk.at[0, 0, 0], sem_u).wait()
            pltpu.make_async_copy(it_hbm.at[pl.ds(0, 8), pl.ds(0, 128)],
                                  iblk.at[0, 0, 0], sem_i).wait()

    def compute_group(g, par):
        idvu, idvi = idvecs(g)
        sl = jnp.full((L,), par, jnp.int32)
        for q in range(GQ):
            l = par * GQ + q
            jru = jnp.full((L,), idvu[l] & 127, jnp.int32)
            jri = jnp.full((L,), idvi[l] & 127, jnp.int32)
            qv = jnp.full((L,), q, jnp.int32)
            u_lo = plsc.load_gather(ublk, [sl, qv, fg_lo, s_v, jru])
            u_hi = plsc.load_gather(ublk, [sl, qv, fg_hi, s_v, jru])
            i_lo = plsc.load_gather(iblk, [sl, qv, fg_lo, s_v, jri])
            i_hi = plsc.load_gather(iblk, [sl, qv, fg_hi, s_v, jri])
            t = (u_lo * i_lo) * w_lo + (u_hi * i_hi) * w_hi
            tbuf[g, pl.ds(q * L, L)] = t

    issue_group(0, 0)

    def step(p, carry):
        g0 = p * 2
        issue_group(g0 + 1, 1)
        drain_group()
        compute_group(g0, 0)

        @pl.when(p < NG // 2 - 1)
        def _():
            issue_group(g0 + 2, 0)

        drain_group()
        compute_group(g0 + 1, 1)
        return carry

    lax.fori_loop(0, NG // 2, step, 0, unroll=False)

    def finish(k, carry):
        rows = k * L + lane
        trow = rows >> 2
        tcol = (rows & 3) * L
        acc = jnp.full((L,), bias, jnp.float32)
        for l in range(L):
            acc = acc + plsc.load_gather(tbuf, [trow, tcol + l])
        o = 1.0 / (1.0 + jnp.exp(-acc))
        outv[pl.ds(pl.multiple_of(k * L, 8), L)] = o
        return carry

    lax.fori_loop(0, BPW // L, finish, 0, unroll=False)

    pltpu.sync_copy(outv, res_hbm.at[wid])


@jax.jit
def _gmf(user_ids, item_ids, ut_t, it_t, wb):
    mesh = plsc.VectorSubcoreMesh(core_axis_name="c", subcore_axis_name="s")
    f = functools.partial(
        pl.kernel,
        mesh=mesh,
        out_type=jax.ShapeDtypeStruct((NW, BPW), jnp.float32),
        scratch_types=[
            pltpu.VMEM((BPW + L,), jnp.int32),
            pltpu.VMEM((BPW + L,), jnp.int32),
            pltpu.VMEM((2, GQ, NFG, 8, 128), jnp.float32),
            pltpu.VMEM((2, GQ, NFG, 8, 128), jnp.float32),
            pltpu.VMEM((48,), jnp.float32),
            pltpu.VMEM((NG, GQ * L), jnp.float32),
            pltpu.VMEM((BPW,), jnp.float32),
            pltpu.SemaphoreType.DMA,
            pltpu.SemaphoreType.DMA,
        ],
        compiler_params=pltpu.CompilerParams(
            use_tc_tiling_on_sc=True, needs_layout_passes=False),
    )(_gmf_body)
    return f(user_ids, item_ids, ut_t, it_t, wb)


def kernel(user_ids, item_ids, user_table, item_table, fc_w, fc_b):
    wb = jnp.concatenate(
        [fc_w.reshape(-1), fc_b.reshape(-1),
         jnp.zeros((48 - D - 1,), jnp.float32)])
    res = _gmf(user_ids.astype(jnp.int32), item_ids.astype(jnp.int32),
               user_table.T, item_table.T, wb)
    return res.reshape(B, 1)


# speedup vs baseline: 3.9931x; 1.0060x over previous
import functools

import jax
import jax.numpy as jnp
from jax import lax
from jax.experimental import pallas as pl
from jax.experimental.pallas import tpu as pltpu
from jax.experimental.pallas import tpu_sc as plsc

B = 16384
D = 32
L = 16
NW = 32
BPW = B // NW
NFG = D // 8
GQ = 4
NG = BPW // GQ


def _gmf_body(uid_hbm, iid_hbm, ut_hbm, it_hbm, wb_hbm, res_hbm,
              uidx, iidx, ublk, iblk, wv, tbuf, outv, sem_u, sem_i):
    nc = 2
    wid = lax.axis_index("s") * nc + lax.axis_index("c")
    base = wid * BPW

    pltpu.sync_copy(uid_hbm.at[pl.ds(base, BPW)], uidx.at[pl.ds(0, BPW)])
    pltpu.sync_copy(iid_hbm.at[pl.ds(base, BPW)], iidx.at[pl.ds(0, BPW)])
    pltpu.sync_copy(wb_hbm, wv)

    lane = lax.broadcasted_iota(jnp.int32, (L,), 0)
    w_lo = wv[pl.ds(0, L)]
    w_hi = wv[pl.ds(L, L)]
    bias = wv[pl.ds(2 * L, L)][0]

    def issue_id(j, slot, q, tab, blk, sem):
        co = pl.multiple_of((j >> 7) * 128, 128)
        pltpu.async_copy(tab.at[:, pl.ds(co, 128)], blk.at[slot, q], sem)

    def idvecs(g):
        bb = pl.multiple_of((g >> 1) * 8, 8)
        return uidx[pl.ds(bb, L)], iidx[pl.ds(bb, L)]

    def issue_group(g, par):
        idvu, idvi = idvecs(g)
        for q in range(GQ):
            l = par * GQ + q
            issue_id(idvu[l], par, q, ut_hbm, ublk, sem_u)
            issue_id(idvi[l], par, q, it_hbm, iblk, sem_i)

    def drain_group():
        for _ in range(GQ):
            pltpu.make_async_copy(ut_hbm.at[:, pl.ds(0, 128)],
                                  ublk.at[0, 0], sem_u).wait()
            pltpu.make_async_copy(it_hbm.at[:, pl.ds(0, 128)],
                                  iblk.at[0, 0], sem_i).wait()

    def compute_group(g, par):
        idvu, idvi = idvecs(g)
        sl = jnp.full((L,), par, jnp.int32)
        for q in range(GQ):
            l = par * GQ + q
            jru = jnp.full((L,), idvu[l] & 127, jnp.int32)
            jri = jnp.full((L,), idvi[l] & 127, jnp.int32)
            qv = jnp.full((L,), q, jnp.int32)
            u_lo = plsc.load_gather(ublk, [sl, qv, lane, jru])
            u_hi = plsc.load_gather(ublk, [sl, qv, lane + L, jru])
            i_lo = plsc.load_gather(iblk, [sl, qv, lane, jri])
            i_hi = plsc.load_gather(iblk, [sl, qv, lane + L, jri])
            t = (u_lo * i_lo) * w_lo + (u_hi * i_hi) * w_hi
            tbuf[g, pl.ds(q * L, L)] = t

    issue_group(0, 0)

    def step(p, carry):
        g0 = p * 2
        issue_group(g0 + 1, 1)
        drain_group()
        compute_group(g0, 0)

        @pl.when(p < NG // 2 - 1)
        def _():
            issue_group(g0 + 2, 0)

        drain_group()
        compute_group(g0 + 1, 1)
        return carry

    lax.fori_loop(0, NG // 2, step, 0, unroll=False)

    def finish(k, carry):
        rows = k * L + lane
        trow = rows >> 2
        tcol = (rows & 3) * L
        acc = jnp.full((L,), bias, jnp.float32)
        for l in range(L):
            acc = acc + plsc.load_gather(tbuf, [trow, tcol + l])
        o = 1.0 / (1.0 + jnp.exp(-acc))
        outv[pl.ds(pl.multiple_of(k * L, 8), L)] = o
        return carry

    lax.fori_loop(0, BPW // L, finish, 0, unroll=False)

    pltpu.sync_copy(outv, res_hbm.at[wid])


@jax.jit
def _gmf(user_ids, item_ids, ut_t, it_t, wb):
    mesh = plsc.VectorSubcoreMesh(core_axis_name="c", subcore_axis_name="s")
    f = functools.partial(
        pl.kernel,
        mesh=mesh,
        out_type=jax.ShapeDtypeStruct((NW, BPW), jnp.float32),
        scratch_types=[
            pltpu.VMEM((BPW + L,), jnp.int32),
            pltpu.VMEM((BPW + L,), jnp.int32),
            pltpu.VMEM((2, GQ, D, 128), jnp.float32),
            pltpu.VMEM((2, GQ, D, 128), jnp.float32),
            pltpu.VMEM((48,), jnp.float32),
            pltpu.VMEM((NG, GQ * L), jnp.float32),
            pltpu.VMEM((BPW,), jnp.float32),
            pltpu.SemaphoreType.DMA,
            pltpu.SemaphoreType.DMA,
        ],
        compiler_params=pltpu.CompilerParams(
            use_tc_tiling_on_sc=True, needs_layout_passes=False),
    )(_gmf_body)
    return f(user_ids, item_ids, ut_t, it_t, wb)


def kernel(user_ids, item_ids, user_table, item_table, fc_w, fc_b):
    wb = jnp.concatenate(
        [fc_w.reshape(-1), fc_b.reshape(-1),
         jnp.zeros((48 - D - 1,), jnp.float32)])
    res = _gmf(user_ids.astype(jnp.int32), item_ids.astype(jnp.int32),
               user_table.T, item_table.T, wb)
    return res.reshape(B, 1)


# speedup vs baseline: 4.3880x; 1.0989x over previous
import functools

import jax
import jax.numpy as jnp
from jax import lax
from jax.experimental import pallas as pl
from jax.experimental.pallas import tpu as pltpu
from jax.experimental.pallas import tpu_sc as plsc

B = 16384
D = 32
L = 16
NW = 32
BPW = B // NW
NFG = D // 8
GQ = 4
NG = BPW // GQ


def _gmf_body(uid_hbm, iid_hbm, ut_hbm, it_hbm, wb_hbm, res_hbm,
              uidx, iidx, ublk, iblk, wv, tbuf, outv, sem_u, sem_i):
    nc = 2
    wid = lax.axis_index("s") * nc + lax.axis_index("c")
    base = wid * BPW

    pltpu.sync_copy(uid_hbm.at[pl.ds(base, BPW)], uidx.at[pl.ds(0, BPW)])
    pltpu.sync_copy(iid_hbm.at[pl.ds(base, BPW)], iidx.at[pl.ds(0, BPW)])
    pltpu.sync_copy(wb_hbm, wv)

    lane = lax.broadcasted_iota(jnp.int32, (L,), 0)
    w_lo = wv[pl.ds(0, L)]
    w_hi = wv[pl.ds(L, L)]
    bias = wv[pl.ds(2 * L, L)][0]

    def issue_id(j, slot, q, tab, blk, sem):
        co = pl.multiple_of((j >> 7) * 128, 128)
        pltpu.async_copy(tab.at[:, pl.ds(co, 128)], blk.at[slot, q], sem)

    def idvecs(g):
        bb = pl.multiple_of((g >> 1) * 8, 8)
        return uidx[pl.ds(bb, L)], iidx[pl.ds(bb, L)]

    def issue_group(g, slot, par):
        idvu, idvi = idvecs(g)
        for q in range(GQ):
            l = par * GQ + q
            issue_id(idvu[l], slot, q, ut_hbm, ublk, sem_u)
            issue_id(idvi[l], slot, q, it_hbm, iblk, sem_i)

    def drain_group():
        for _ in range(GQ):
            pltpu.make_async_copy(ut_hbm.at[:, pl.ds(0, 128)],
                                  ublk.at[0, 0], sem_u).wait()
            pltpu.make_async_copy(it_hbm.at[:, pl.ds(0, 128)],
                                  iblk.at[0, 0], sem_i).wait()

    def compute_group(g, slot, par):
        idvu, idvi = idvecs(g)
        sl = jnp.full((L,), slot, jnp.int32)
        for q in range(GQ):
            l = par * GQ + q
            jru = jnp.full((L,), idvu[l] & 127, jnp.int32)
            jri = jnp.full((L,), idvi[l] & 127, jnp.int32)
            qv = jnp.full((L,), q, jnp.int32)
            u_lo = plsc.load_gather(ublk, [sl, qv, lane, jru])
            u_hi = plsc.load_gather(ublk, [sl, qv, lane + L, jru])
            i_lo = plsc.load_gather(iblk, [sl, qv, lane, jri])
            i_hi = plsc.load_gather(iblk, [sl, qv, lane + L, jri])
            t = (u_lo * i_lo) * w_lo + (u_hi * i_hi) * w_hi
            tbuf[g, pl.ds(q * L, L)] = t

    issue_group(0, 0, 0)
    issue_group(1, 1, 1)

    def step(tp, carry):
        g0 = tp * 6
        for k in range(6):
            tgt = g0 + k + 2

            @pl.when(tgt < NG)
            def _(tgt=tgt, k=k):
                issue_group(tgt, (k + 2) % 3, k & 1)

            drain_group()
            compute_group(g0 + k, k % 3, k & 1)
        return carry

    lax.fori_loop(0, NG // 6, step, 0, unroll=False)
    for g, k in ((126, 0), (127, 1)):
        drain_group()
        compute_group(g, k % 3, k & 1)

    def finish(k, carry):
        rows = k * L + lane
        trow = rows >> 2
        tcol = (rows & 3) * L
        acc = jnp.full((L,), bias, jnp.float32)
        for l in range(L):
            acc = acc + plsc.load_gather(tbuf, [trow, tcol + l])
        o = 1.0 / (1.0 + jnp.exp(-acc))
        outv[pl.ds(pl.multiple_of(k * L, 8), L)] = o
        return carry

    lax.fori_loop(0, BPW // L, finish, 0, unroll=False)

    pltpu.sync_copy(outv, res_hbm.at[wid])


@jax.jit
def _gmf(user_ids, item_ids, ut_t, it_t, wb):
    mesh = plsc.VectorSubcoreMesh(core_axis_name="c", subcore_axis_name="s")
    f = functools.partial(
        pl.kernel,
        mesh=mesh,
        out_type=jax.ShapeDtypeStruct((NW, BPW), jnp.float32),
        scratch_types=[
            pltpu.VMEM((BPW + L,), jnp.int32),
            pltpu.VMEM((BPW + L,), jnp.int32),
            pltpu.VMEM((3, GQ, D, 128), jnp.float32),
            pltpu.VMEM((3, GQ, D, 128), jnp.float32),
            pltpu.VMEM((48,), jnp.float32),
            pltpu.VMEM((NG, GQ * L), jnp.float32),
            pltpu.VMEM((BPW,), jnp.float32),
            pltpu.SemaphoreType.DMA,
            pltpu.SemaphoreType.DMA,
        ],
        compiler_params=pltpu.CompilerParams(
            use_tc_tiling_on_sc=True, needs_layout_passes=False),
    )(_gmf_body)
    return f(user_ids, item_ids, ut_t, it_t, wb)


def kernel(user_ids, item_ids, user_table, item_table, fc_w, fc_b):
    wb = jnp.concatenate(
        [fc_w.reshape(-1), fc_b.reshape(-1),
         jnp.zeros((48 - D - 1,), jnp.float32)])
    res = _gmf(user_ids.astype(jnp.int32), item_ids.astype(jnp.int32),
               user_table.T, item_table.T, wb)
    return res.reshape(B, 1)
